# Initial kernel scaffold; baseline (speedup 1.0000x reference)
import jax
import jax.numpy as jnp
from jax.experimental import pallas as pl


def kernel(real_obs, action_mask, W1, a1_src, a1_dst, W2, a2_src, a2_dst, P1w, P1b, P2w, P2b):
    raise NotImplementedError("write your pallas kernel here")


# speedup vs baseline: 1698.6459x; 1698.6459x over previous
import functools

import jax
import jax.numpy as jnp
from jax import lax
from jax.experimental import pallas as pl

HEADS = 3
NEG_SLOPE = 0.2


def _leaky_relu(x):
    return jnp.where(x >= 0, x, NEG_SLOPE * x)


def _elu(x):
    return jnp.where(x > 0, x, jnp.exp(jnp.minimum(x, 0.0)) - 1.0)


def _row_softmax(e):
    m = jnp.max(e, axis=1, keepdims=True)
    ee = jnp.exp(e - m)
    den = jnp.sum(ee, axis=1, keepdims=True)
    return ee / (den + 1e-16)


def _gat_dense(h, a_src_row, a_dst_row):
    ad_col = jnp.sum(h * a_dst_row, axis=1, keepdims=True)
    as_row = lax.dot_general(a_src_row, h, (((1,), (1,)), ((), ())),
                             preferred_element_type=jnp.float32)
    e = _leaky_relu(ad_col + as_row)
    attn = _row_softmax(e)
    return jnp.dot(attn, h, preferred_element_type=jnp.float32)


def _gnn_kernel(x_ref, w1_ref, a1s_ref, a1d_ref, w2_ref, a2s_ref, a2d_ref,
                p1w_ref, p1b_ref, p2w_ref, p2b_ref, out_ref, *, n, jobs,
                batch_pad):
    x = x_ref[...]

    head_outs = []
    for h in range(HEADS):
        w = w1_ref[h]
        hfeat = jnp.dot(x, w, preferred_element_type=jnp.float32)
        a_s = a1s_ref[pl.ds(h, 1), :]
        a_d = a1d_ref[pl.ds(h, 1), :]
        head_outs.append(_gat_dense(hfeat, a_s, a_d))
    h1 = _elu(jnp.concatenate(head_outs, axis=1))

    h2feat = jnp.dot(h1, w2_ref[...], preferred_element_type=jnp.float32)
    h2 = _elu(_gat_dense(h2feat, a2s_ref[...], a2d_ref[...]))

    f1 = jnp.maximum(
        jnp.dot(h2, p1w_ref[...], preferred_element_type=jnp.float32)
        + p1b_ref[...], 0.0)
    f2 = jnp.maximum(
        jnp.dot(f1, p2w_ref[...], preferred_element_type=jnp.float32)
        + p2b_ref[...], 0.0)

    row_b = lax.broadcasted_iota(jnp.int32, (batch_pad, n), 0)
    col_n = lax.broadcasted_iota(jnp.int32, (batch_pad, n), 1)
    lo = row_b * jobs
    sel = jnp.where((col_n >= lo) & (col_n < lo + jobs), 1.0 / jobs, 0.0)
    out_ref[...] = jnp.dot(sel, f2, preferred_element_type=jnp.float32)


@jax.jit
def kernel(real_obs, action_mask, W1, a1_src, a1_dst, W2, a2_src, a2_dst,
           P1w, P1b, P2w, P2b):
    B, J, F = real_obs.shape
    N = B * J
    HID = P2w.shape[1]
    flat = real_obs.reshape(N, F)
    batch_pad = 8

    body = functools.partial(_gnn_kernel, n=N, jobs=J, batch_pad=batch_pad)
    feats8 = pl.pallas_call(
        body,
        out_shape=jax.ShapeDtypeStruct((batch_pad, HID), jnp.float32),
    )(flat, W1, a1_src, a1_dst, W2,
      a2_src.reshape(1, -1), a2_dst.reshape(1, -1),
      P1w, P1b.reshape(1, -1), P2w, P2b.reshape(1, -1))
    return feats8[:B], action_mask


# speedup vs baseline: 1731.6736x; 1.0194x over previous
import functools

import jax
import jax.numpy as jnp
from jax import lax
from jax.experimental import pallas as pl

HEADS = 3
NEG_SLOPE = 0.2


def _leaky_relu(x):
    return jnp.where(x >= 0, x, NEG_SLOPE * x)


def _elu(x):
    return jnp.where(x > 0, x, jnp.exp(x) - 1.0)


def _gat_dense(h, a_src_row, a_dst_row):
    d = h.shape[1]
    ad_col = jnp.sum(h * a_dst_row, axis=1, keepdims=True)
    as_row = lax.dot_general(a_src_row, h, (((1,), (1,)), ((), ())),
                             preferred_element_type=jnp.float32)
    as_max = jnp.max(as_row, axis=1, keepdims=True)
    e = _leaky_relu(ad_col + as_row)
    emax = _leaky_relu(ad_col + as_max)
    ee = jnp.exp(e - emax)
    h_aug = jnp.concatenate([h, jnp.ones_like(h[:, :1])], axis=1)
    agg = jnp.dot(ee, h_aug, preferred_element_type=jnp.float32)
    return agg[:, :d] / (agg[:, d:d + 1] + 1e-16)


def _gnn_kernel(x_ref, w1_ref, a1s_ref, a1d_ref, w2_ref, a2s_ref, a2d_ref,
                p1w_ref, p1b_ref, p2w_ref, p2b_ref, out_ref, *, n, jobs,
                batch_pad):
    x = x_ref[...]

    head_outs = []
    for h in range(HEADS):
        w = w1_ref[h]
        hfeat = jnp.dot(x, w, preferred_element_type=jnp.float32)
        a_s = a1s_ref[pl.ds(h, 1), :]
        a_d = a1d_ref[pl.ds(h, 1), :]
        head_outs.append(_gat_dense(hfeat, a_s, a_d))
    h1 = _elu(jnp.concatenate(head_outs, axis=1))

    h2feat = jnp.dot(h1, w2_ref[...], preferred_element_type=jnp.float32)
    h2 = _elu(_gat_dense(h2feat, a2s_ref[...], a2d_ref[...]))

    f1 = jnp.maximum(
        jnp.dot(h2, p1w_ref[...], preferred_element_type=jnp.float32)
        + p1b_ref[...], 0.0)
    f2 = jnp.maximum(
        jnp.dot(f1, p2w_ref[...], preferred_element_type=jnp.float32)
        + p2b_ref[...], 0.0)

    row_b = lax.broadcasted_iota(jnp.int32, (batch_pad, n), 0)
    col_n = lax.broadcasted_iota(jnp.int32, (batch_pad, n), 1)
    lo = row_b * jobs
    sel = jnp.where((col_n >= lo) & (col_n < lo + jobs), 1.0 / jobs, 0.0)
    out_ref[...] = jnp.dot(sel, f2, preferred_element_type=jnp.float32)


@jax.jit
def kernel(real_obs, action_mask, W1, a1_src, a1_dst, W2, a2_src, a2_dst,
           P1w, P1b, P2w, P2b):
    B, J, F = real_obs.shape
    N = B * J
    HID = P2w.shape[1]
    flat = real_obs.reshape(N, F)
    batch_pad = 8

    body = functools.partial(_gnn_kernel, n=N, jobs=J, batch_pad=batch_pad)
    feats8 = pl.pallas_call(
        body,
        out_shape=jax.ShapeDtypeStruct((batch_pad, HID), jnp.float32),
    )(flat, W1, a1_src, a1_dst, W2,
      a2_src.reshape(1, -1), a2_dst.reshape(1, -1),
      P1w, P1b.reshape(1, -1), P2w, P2b.reshape(1, -1))
    return feats8[:B], action_mask


# speedup vs baseline: 2070.2662x; 1.1955x over previous
import functools

import jax
import jax.numpy as jnp
from jax import lax
from jax.experimental import pallas as pl

HEADS = 3
NEG_SLOPE = 0.2


def _leaky_relu(x):
    return jnp.where(x >= 0, x, NEG_SLOPE * x)


def _elu(x):
    return jnp.where(x > 0, x, jnp.exp(x) - 1.0)


def _gat_dense(h, a_src_row, a_dst_row):
    d = h.shape[1]
    ad_col = jnp.sum(h * a_dst_row, axis=1, keepdims=True)
    as_row = lax.dot_general(a_src_row, h, (((1,), (1,)), ((), ())),
                             preferred_element_type=jnp.float32)
    as_max = jnp.max(as_row, axis=1, keepdims=True)
    e = _leaky_relu(ad_col + as_row)
    emax = _leaky_relu(ad_col + as_max)
    ee = jnp.exp(e - emax)
    h_aug = jnp.concatenate([h, jnp.ones_like(h[:, :1])], axis=1)
    agg = jnp.dot(ee, h_aug, preferred_element_type=jnp.float32)
    return agg[:, :d] / (agg[:, d:d + 1] + 1e-16)


def _gnn_kernel(x_ref, mask_ref, w1_ref, a1s_ref, a1d_ref, w2_ref, a2s_ref,
                a2d_ref, p1w_ref, p1b_ref, p2w_ref, p2b_ref, out_ref,
                mask_out_ref, *, n, jobs):
    batch = x_ref.shape[0]
    x = jnp.concatenate([x_ref[b] for b in range(batch)], axis=0)

    head_outs = []
    for h in range(HEADS):
        w = w1_ref[h]
        hfeat = jnp.dot(x, w, preferred_element_type=jnp.float32)
        a_s = a1s_ref[pl.ds(h, 1), :]
        a_d = a1d_ref[pl.ds(h, 1), :]
        head_outs.append(_gat_dense(hfeat, a_s, a_d))
    h1 = _elu(jnp.concatenate(head_outs, axis=1))

    h2feat = jnp.dot(h1, w2_ref[...], preferred_element_type=jnp.float32)
    h2 = _elu(_gat_dense(h2feat, a2s_ref[...], a2d_ref[...]))

    f1 = jnp.maximum(
        jnp.dot(h2, p1w_ref[...], preferred_element_type=jnp.float32)
        + p1b_ref[...], 0.0)
    f2 = jnp.maximum(
        jnp.dot(f1, p2w_ref[...], preferred_element_type=jnp.float32)
        + p2b_ref[...], 0.0)

    row_b = lax.broadcasted_iota(jnp.int32, (batch, n), 0)
    col_n = lax.broadcasted_iota(jnp.int32, (batch, n), 1)
    lo = row_b * jobs
    sel = jnp.where((col_n >= lo) & (col_n < lo + jobs), 1.0 / jobs, 0.0)
    out_ref[...] = jnp.dot(sel, f2, preferred_element_type=jnp.float32)
    mask_out_ref[...] = mask_ref[...]


@jax.jit
def kernel(real_obs, action_mask, W1, a1_src, a1_dst, W2, a2_src, a2_dst,
           P1w, P1b, P2w, P2b):
    B, J, F = real_obs.shape
    N = B * J
    HID = P2w.shape[1]

    body = functools.partial(_gnn_kernel, n=N, jobs=J)
    feats, mask_out = pl.pallas_call(
        body,
        out_shape=(jax.ShapeDtypeStruct((B, HID), jnp.float32),
                   jax.ShapeDtypeStruct((B, J), action_mask.dtype)),
    )(real_obs, action_mask, W1, a1_src, a1_dst, W2,
      a2_src.reshape(1, -1), a2_dst.reshape(1, -1),
      P1w, P1b.reshape(1, -1), P2w, P2b.reshape(1, -1))
    return feats, mask_out
